# baseline (speedup 1.0000x reference)
import functools

import jax
import jax.numpy as jnp
from jax import lax
from jax.experimental import pallas as pl
from jax.experimental.pallas import tpu as pltpu
from jax.experimental.pallas import tpu_sc as plsc

N = 10000
E = 320000
F = 128
C_OUT = 2

NC = 2
NS = 16
NW = NC * NS
EPT = E // NW
CHUNK = 125
NCHUNK = EPT // CHUNK
N_PAD = 10240
RPS = N_PAD // NS
ZCHUNK = 128
ZCOPIES = RPS // ZCHUNK

DEG_L = 16

_MESH = plsc.VectorSubcoreMesh(core_axis_name="c", subcore_axis_name="s")


@functools.partial(
    pl.kernel,
    out_type=jax.ShapeDtypeStruct((NC, N_PAD, DEG_L), jnp.float32),
    mesh=_MESH,
    compiler_params=pltpu.CompilerParams(use_tc_tiling_on_sc=False),
    scratch_types=[
        pltpu.VMEM((NCHUNK, CHUNK), jnp.int32),
        pltpu.VMEM((CHUNK, DEG_L), jnp.float32),
        pltpu.VMEM((ZCHUNK, DEG_L), jnp.float32),
        pltpu.VMEM_SHARED((N_PAD, DEG_L), jnp.float32),
    ],
)
def _deg_sc(dst_hbm, deg_hbm, dst_v, ones_v, zbuf_v, acc):
    cid = lax.axis_index("c")
    sid = lax.axis_index("s")
    wid = sid * NC + cid

    @pl.loop(0, CHUNK)
    def _(r):
        ones_v[r, :] = jnp.full((DEG_L,), 1.0, jnp.float32)

    @pl.loop(0, ZCHUNK)
    def _(r):
        zbuf_v[r, :] = jnp.zeros((DEG_L,), jnp.float32)

    @pl.loop(0, ZCOPIES)
    def _(k):
        pltpu.sync_copy(zbuf_v, acc.at[pl.ds(sid * RPS + k * ZCHUNK, ZCHUNK)])

    plsc.subcore_barrier()

    pltpu.sync_copy(dst_hbm.at[wid], dst_v)

    @pl.loop(0, NCHUNK)
    def _(j):
        pltpu.sync_copy(ones_v, acc.at[dst_v.at[j]], add=True)

    plsc.subcore_barrier()
    pltpu.sync_copy(acc.at[pl.ds(sid * RPS, RPS)],
                    deg_hbm.at[cid, pl.ds(sid * RPS, RPS)])


FH = F // 2
NCHUNK2 = E // NS // CHUNK


NBUF = 5


@functools.partial(
    pl.kernel,
    out_type=(jax.ShapeDtypeStruct((N_PAD, FH), jnp.float32),
              jax.ShapeDtypeStruct((N_PAD, FH), jnp.float32)),
    mesh=_MESH,
    compiler_params=pltpu.CompilerParams(use_tc_tiling_on_sc=False),
    scratch_types=[
        pltpu.VMEM((NCHUNK2, CHUNK), jnp.int32),
        pltpu.VMEM((NCHUNK2, CHUNK), jnp.int32),
        pltpu.VMEM((NBUF, CHUNK, FH), jnp.float32),
        pltpu.VMEM((ZCHUNK, FH), jnp.float32),
        pltpu.VMEM_SHARED((N_PAD, FH), jnp.float32),
        pltpu.SemaphoreType.DMA,
        pltpu.SemaphoreType.DMA,
    ],
)
def _agg_sc(xs_lo_hbm, xs_hi_hbm, src_hbm, dst_hbm, agg_lo_hbm, agg_hi_hbm,
            src_v, dst_v, buf_v, zbuf_v, acc, sem_g, sem_s):
    cid = lax.axis_index("c")
    sid = lax.axis_index("s")

    @pl.loop(0, ZCHUNK)
    def _(r):
        @pl.loop(0, FH // 16)
        def _(col):
            zbuf_v[r, pl.ds(col * 16, 16)] = jnp.zeros((16,), jnp.float32)

    @pl.loop(0, ZCOPIES)
    def _(k):
        pltpu.sync_copy(zbuf_v, acc.at[pl.ds(sid * RPS + k * ZCHUNK, ZCHUNK)])

    pltpu.sync_copy(src_hbm.at[sid], src_v)
    pltpu.sync_copy(dst_hbm.at[sid], dst_v)
    plsc.subcore_barrier()

    def run(xs_hbm, out_hbm):
        @pl.loop(0, NCHUNK2 // NBUF)
        def _(j8):
            c0 = j8 * NBUF
            for k in range(NBUF):
                @pl.when(j8 > 0)
                def _():
                    pltpu.make_async_copy(
                        buf_v.at[k], acc.at[dst_v.at[c0 + k]], sem_s).wait()
                pltpu.async_copy(xs_hbm.at[src_v.at[c0 + k]], buf_v.at[k], sem_g)
            for k in range(NBUF):
                pltpu.make_async_copy(
                    xs_hbm.at[src_v.at[c0 + k]], buf_v.at[k], sem_g).wait()
                pltpu.async_copy(buf_v.at[k], acc.at[dst_v.at[c0 + k]], sem_s,
                                 add=True)

        for k in range(NBUF):
            pltpu.make_async_copy(buf_v.at[k], acc.at[dst_v.at[k]], sem_s).wait()

        plsc.subcore_barrier()
        pltpu.sync_copy(acc.at[pl.ds(sid * RPS, RPS)],
                        out_hbm.at[pl.ds(sid * RPS, RPS)])

    @pl.when(cid == 0)
    def _():
        run(xs_lo_hbm, agg_lo_hbm)

    @pl.when(cid == 1)
    def _():
        run(xs_hi_hbm, agg_hi_hbm)


BLK = 1000


def _scale_body(nodes_ref, d0_ref, d1_ref, lo_ref, hi_ref):
    d = d0_ref[:, 0:1] + d1_ref[:, 0:1]
    dinv = jnp.where(d > 0, lax.rsqrt(d), 0.0)
    xs = nodes_ref[...] * dinv
    lo_ref[...] = xs[:, :FH]
    hi_ref[...] = xs[:, FH:]


def _scale_call(nodes, d0, d1):
    return pl.pallas_call(
        _scale_body,
        grid=(N // BLK,),
        in_specs=[
            pl.BlockSpec((BLK, F), lambda i: (i, 0)),
            pl.BlockSpec((BLK, DEG_L), lambda i: (i, 0)),
            pl.BlockSpec((BLK, DEG_L), lambda i: (i, 0)),
        ],
        out_specs=[pl.BlockSpec((BLK, FH), lambda i: (i, 0)),
                   pl.BlockSpec((BLK, FH), lambda i: (i, 0))],
        out_shape=[jax.ShapeDtypeStruct((N, FH), jnp.float32),
                   jax.ShapeDtypeStruct((N, FH), jnp.float32)],
    )(nodes, d0, d1)


def _dot(a, b):
    return jnp.dot(a, b, preferred_element_type=jnp.float32)


def _mlp_body(nodes_ref, alo_ref, ahi_ref, d0_ref,
              d1_ref, convWt_ref, convb_ref, fc1Wt_ref, fc1b_ref, fc2Wt_ref,
              fc2b_ref, out_ref):
    d = d0_ref[:, 0:1] + d1_ref[:, 0:1]
    dinv = jnp.where(d > 0, lax.rsqrt(d), 0.0)
    agg_lo = alo_ref[...] * dinv
    agg_hi = ahi_ref[...] * dinv
    conv = (_dot(agg_lo, convWt_ref[:FH, :]) + _dot(agg_hi, convWt_ref[FH:, :])
            + convb_ref[...])
    x = nodes_ref[...]
    z = _dot(x, fc1Wt_ref[:F, :]) + _dot(conv, fc1Wt_ref[F:, :]) + fc1b_ref[...]
    z = jnp.maximum(z, 0.0)
    logits = _dot(z, fc2Wt_ref[...]) + fc2b_ref[...]
    m = jnp.max(logits, axis=1, keepdims=True)
    lse = m + jnp.log(jnp.sum(jnp.exp(logits - m), axis=1, keepdims=True))
    out_ref[...] = logits - lse


def _mlp_call(nodes, alo, ahi, d0, d1, convWt, convb, fc1Wt,
              fc1b, fc2Wt, fc2b):
    full = lambda shape: pl.BlockSpec(shape, lambda i: (0, 0))
    agg_spec = pl.BlockSpec((BLK, FH), lambda i: (i, 0))
    return pl.pallas_call(
        _mlp_body,
        grid=(N // BLK,),
        in_specs=[
            pl.BlockSpec((BLK, F), lambda i: (i, 0)),
            agg_spec, agg_spec,
            pl.BlockSpec((BLK, DEG_L), lambda i: (i, 0)),
            pl.BlockSpec((BLK, DEG_L), lambda i: (i, 0)),
            full((F, F)),
            full((1, F)),
            full((2 * F, F)),
            full((1, F)),
            full((F, C_OUT)),
            full((1, C_OUT)),
        ],
        out_specs=pl.BlockSpec((BLK, C_OUT), lambda i: (i, 0)),
        out_shape=jax.ShapeDtypeStruct((N, C_OUT), jnp.float32),
    )(nodes, alo, ahi, d0, d1, convWt, convb, fc1Wt, fc1b,
      fc2Wt, fc2b)


def kernel(nodes, edge_index, conv_W, conv_b, fc1_W, fc1_b, fc2_W, fc2_b):
    dst32 = edge_index[1].reshape(NW, NCHUNK, CHUNK)
    src16 = edge_index[0].reshape(NS, NCHUNK2, CHUNK)
    dst16 = edge_index[1].reshape(NS, NCHUNK2, CHUNK)

    deg2 = _deg_sc(dst32)
    d0, d1 = deg2[0], deg2[1]

    xs_lo, xs_hi = _scale_call(nodes, d0, d1)
    agg_lo, agg_hi = _agg_sc(xs_lo, xs_hi, src16, dst16)

    return _mlp_call(
        nodes, agg_lo, agg_hi, d0, d1,
        conv_W.T, conv_b.reshape(1, F),
        fc1_W.T, fc1_b.reshape(1, F),
        fc2_W.T, fc2_b.reshape(1, C_OUT),
    )


# speedup vs baseline: 1.0256x; 1.0256x over previous
import functools

import jax
import jax.numpy as jnp
from jax import lax
from jax.experimental import pallas as pl
from jax.experimental.pallas import tpu as pltpu
from jax.experimental.pallas import tpu_sc as plsc

N = 10000
E = 320000
F = 128
C_OUT = 2

NC = 2
NS = 16
NW = NC * NS
EPT = E // NW
CHUNK = 125
NCHUNK = EPT // CHUNK
N_PAD = 10240
RPS = N_PAD // NS
ZCHUNK = 128
ZCOPIES = RPS // ZCHUNK

DEG_L = 16

_MESH = plsc.VectorSubcoreMesh(core_axis_name="c", subcore_axis_name="s")


@functools.partial(
    pl.kernel,
    out_type=jax.ShapeDtypeStruct((NC, N_PAD, DEG_L), jnp.float32),
    mesh=_MESH,
    compiler_params=pltpu.CompilerParams(use_tc_tiling_on_sc=False),
    scratch_types=[
        pltpu.VMEM((NCHUNK, CHUNK), jnp.int32),
        pltpu.VMEM((CHUNK, DEG_L), jnp.float32),
        pltpu.VMEM((ZCHUNK, DEG_L), jnp.float32),
        pltpu.VMEM_SHARED((N_PAD, DEG_L), jnp.float32),
    ],
)
def _deg_sc(dst_hbm, deg_hbm, dst_v, ones_v, zbuf_v, acc):
    cid = lax.axis_index("c")
    sid = lax.axis_index("s")
    wid = sid * NC + cid

    @pl.loop(0, CHUNK)
    def _(r):
        ones_v[r, :] = jnp.full((DEG_L,), 1.0, jnp.float32)

    @pl.loop(0, ZCHUNK)
    def _(r):
        zbuf_v[r, :] = jnp.zeros((DEG_L,), jnp.float32)

    @pl.loop(0, ZCOPIES)
    def _(k):
        pltpu.sync_copy(zbuf_v, acc.at[pl.ds(sid * RPS + k * ZCHUNK, ZCHUNK)])

    plsc.subcore_barrier()

    pltpu.sync_copy(dst_hbm.at[wid], dst_v)

    @pl.loop(0, NCHUNK)
    def _(j):
        pltpu.sync_copy(ones_v, acc.at[dst_v.at[j]], add=True)

    plsc.subcore_barrier()
    pltpu.sync_copy(acc.at[pl.ds(sid * RPS, RPS)],
                    deg_hbm.at[cid, pl.ds(sid * RPS, RPS)])


FH = F // 2
NCHUNK2 = E // NS // CHUNK


NBUF = 5


@functools.partial(
    pl.kernel,
    out_type=(jax.ShapeDtypeStruct((N_PAD, FH), jnp.float32),
              jax.ShapeDtypeStruct((N_PAD, FH), jnp.float32)),
    mesh=_MESH,
    compiler_params=pltpu.CompilerParams(use_tc_tiling_on_sc=False),
    scratch_types=[
        pltpu.VMEM((NCHUNK2, CHUNK), jnp.int32),
        pltpu.VMEM((NCHUNK2, CHUNK), jnp.int32),
        pltpu.VMEM((NBUF, CHUNK, FH), jnp.float32),
        pltpu.VMEM((ZCHUNK, FH), jnp.float32),
        pltpu.VMEM_SHARED((N_PAD, FH), jnp.float32),
        pltpu.SemaphoreType.DMA,
        pltpu.SemaphoreType.DMA,
    ],
)
def _agg_sc(xs_lo_hbm, xs_hi_hbm, src_hbm, dst_hbm, agg_lo_hbm, agg_hi_hbm,
            src_v, dst_v, buf_v, zbuf_v, acc, sem_g, sem_s):
    cid = lax.axis_index("c")
    sid = lax.axis_index("s")

    @pl.loop(0, ZCHUNK)
    def _(r):
        @pl.loop(0, FH // 16)
        def _(col):
            zbuf_v[r, pl.ds(col * 16, 16)] = jnp.zeros((16,), jnp.float32)

    @pl.loop(0, ZCOPIES)
    def _(k):
        pltpu.sync_copy(zbuf_v, acc.at[pl.ds(sid * RPS + k * ZCHUNK, ZCHUNK)])

    pltpu.sync_copy(src_hbm.at[sid], src_v)
    pltpu.sync_copy(dst_hbm.at[sid], dst_v)
    plsc.subcore_barrier()

    def run(xs_hbm, out_hbm):
        @pl.loop(0, NCHUNK2 // NBUF)
        def _(j8):
            c0 = j8 * NBUF
            for k in range(NBUF):
                @pl.when(j8 > 0)
                def _():
                    pltpu.make_async_copy(
                        buf_v.at[k], acc.at[dst_v.at[c0 + k]], sem_s).wait()
                pltpu.async_copy(xs_hbm.at[src_v.at[c0 + k]], buf_v.at[k], sem_g)
            for k in range(NBUF):
                pltpu.make_async_copy(
                    xs_hbm.at[src_v.at[c0 + k]], buf_v.at[k], sem_g).wait()
                pltpu.async_copy(buf_v.at[k], acc.at[dst_v.at[c0 + k]], sem_s,
                                 add=True)

        for k in range(NBUF):
            pltpu.make_async_copy(buf_v.at[k], acc.at[dst_v.at[k]], sem_s).wait()

        plsc.subcore_barrier()
        pltpu.sync_copy(acc.at[pl.ds(sid * RPS, RPS)],
                        out_hbm.at[pl.ds(sid * RPS, RPS)])

    @pl.when(cid == 0)
    def _():
        run(xs_lo_hbm, agg_lo_hbm)

    @pl.when(cid == 1)
    def _():
        run(xs_hi_hbm, agg_hi_hbm)


BLK = 2000


def _scale_body(nodes_ref, d0_ref, d1_ref, lo_ref, hi_ref):
    d = d0_ref[:, 0:1] + d1_ref[:, 0:1]
    dinv = jnp.where(d > 0, lax.rsqrt(d), 0.0)
    xs = nodes_ref[...] * dinv
    lo_ref[...] = xs[:, :FH]
    hi_ref[...] = xs[:, FH:]


def _scale_call(nodes, d0, d1):
    return pl.pallas_call(
        _scale_body,
        grid=(N // BLK,),
        in_specs=[
            pl.BlockSpec((BLK, F), lambda i: (i, 0)),
            pl.BlockSpec((BLK, DEG_L), lambda i: (i, 0)),
            pl.BlockSpec((BLK, DEG_L), lambda i: (i, 0)),
        ],
        out_specs=[pl.BlockSpec((BLK, FH), lambda i: (i, 0)),
                   pl.BlockSpec((BLK, FH), lambda i: (i, 0))],
        out_shape=[jax.ShapeDtypeStruct((N, FH), jnp.float32),
                   jax.ShapeDtypeStruct((N, FH), jnp.float32)],
    )(nodes, d0, d1)


def _dot(a, b):
    return jnp.dot(a, b, preferred_element_type=jnp.float32)


def _mlp_body(nodes_ref, alo_ref, ahi_ref, d0_ref,
              d1_ref, convWt_ref, convb_ref, fc1Wt_ref, fc1b_ref, fc2Wt_ref,
              fc2b_ref, out_ref):
    d = d0_ref[:, 0:1] + d1_ref[:, 0:1]
    dinv = jnp.where(d > 0, lax.rsqrt(d), 0.0)
    agg_lo = alo_ref[...] * dinv
    agg_hi = ahi_ref[...] * dinv
    conv = (_dot(agg_lo, convWt_ref[:FH, :]) + _dot(agg_hi, convWt_ref[FH:, :])
            + convb_ref[...])
    x = nodes_ref[...]
    z = _dot(x, fc1Wt_ref[:F, :]) + _dot(conv, fc1Wt_ref[F:, :]) + fc1b_ref[...]
    z = jnp.maximum(z, 0.0)
    logits = _dot(z, fc2Wt_ref[...]) + fc2b_ref[...]
    m = jnp.max(logits, axis=1, keepdims=True)
    lse = m + jnp.log(jnp.sum(jnp.exp(logits - m), axis=1, keepdims=True))
    out_ref[...] = logits - lse


def _mlp_call(nodes, alo, ahi, d0, d1, convWt, convb, fc1Wt,
              fc1b, fc2Wt, fc2b):
    full = lambda shape: pl.BlockSpec(shape, lambda i: (0, 0))
    agg_spec = pl.BlockSpec((BLK, FH), lambda i: (i, 0))
    return pl.pallas_call(
        _mlp_body,
        grid=(N // BLK,),
        in_specs=[
            pl.BlockSpec((BLK, F), lambda i: (i, 0)),
            agg_spec, agg_spec,
            pl.BlockSpec((BLK, DEG_L), lambda i: (i, 0)),
            pl.BlockSpec((BLK, DEG_L), lambda i: (i, 0)),
            full((F, F)),
            full((1, F)),
            full((2 * F, F)),
            full((1, F)),
            full((F, C_OUT)),
            full((1, C_OUT)),
        ],
        out_specs=pl.BlockSpec((BLK, C_OUT), lambda i: (i, 0)),
        out_shape=jax.ShapeDtypeStruct((N, C_OUT), jnp.float32),
    )(nodes, alo, ahi, d0, d1, convWt, convb, fc1Wt, fc1b,
      fc2Wt, fc2b)


def kernel(nodes, edge_index, conv_W, conv_b, fc1_W, fc1_b, fc2_W, fc2_b):
    dst32 = edge_index[1].reshape(NW, NCHUNK, CHUNK)
    src16 = edge_index[0].reshape(NS, NCHUNK2, CHUNK)
    dst16 = edge_index[1].reshape(NS, NCHUNK2, CHUNK)

    deg2 = _deg_sc(dst32)
    d0, d1 = deg2[0], deg2[1]

    xs_lo, xs_hi = _scale_call(nodes, d0, d1)
    agg_lo, agg_hi = _agg_sc(xs_lo, xs_hi, src16, dst16)

    return _mlp_call(
        nodes, agg_lo, agg_hi, d0, d1,
        conv_W.T, conv_b.reshape(1, F),
        fc1_W.T, fc1_b.reshape(1, F),
        fc2_W.T, fc2_b.reshape(1, C_OUT),
    )
